# Initial kernel scaffold; baseline (speedup 1.0000x reference)
import jax
import jax.numpy as jnp
from jax.experimental import pallas as pl


def kernel(x_cat, x_num, tables, W1, b1, g1, beta1, W2, b2, g2, beta2, W3, b3):
    raise NotImplementedError("write your pallas kernel here")


# speedup vs baseline: 28.5681x; 28.5681x over previous
import functools

import jax
import jax.numpy as jnp
from jax import lax
from jax.experimental import pallas as pl
from jax.experimental.pallas import tpu as pltpu
from jax.experimental.pallas import tpu_sc as plsc

B = 16384
F = 26
V = 1000
D = V // 2
NUM = 13
H1, H2 = 256, 128
EPS = 1e-5

NC, NS, L = 2, 16, 16
NW = NC * NS
BPW = B // NW
CH = 4
CHF = CH * F
NCH = BPW // CH
OB = 64
OBJ = OB // CH
NV = H1 // L


def _proj_body(t_ref, w_ref, o_ref):
    o_ref[0] = jnp.dot(t_ref[0], w_ref[0], preferred_element_type=jnp.float32)


def _project_tables(tables, w1_emb):
    return pl.pallas_call(
        _proj_body,
        grid=(F,),
        in_specs=[
            pl.BlockSpec((1, V, D), lambda f: (f, 0, 0)),
            pl.BlockSpec((1, D, H1), lambda f: (f, 0, 0)),
        ],
        out_specs=pl.BlockSpec((1, V, H1), lambda f: (f, 0, 0)),
        out_shape=jax.ShapeDtypeStruct((F, V, H1), jnp.float32),
    )(tables, w1_emb)


def _sc_embed_bag(p_flat, idx3):
    mesh = plsc.VectorSubcoreMesh(core_axis_name="c", subcore_axis_name="s")

    @functools.partial(
        pl.kernel,
        out_type=jax.ShapeDtypeStruct((B, H1), jnp.float32),
        mesh=mesh,
        scratch_types=[
            pltpu.VMEM((NCH, CHF), jnp.int32),
            pltpu.VMEM((CHF, H1), jnp.float32),
            pltpu.VMEM((CHF, H1), jnp.float32),
            pltpu.VMEM((OB, H1), jnp.float32),
            pltpu.SemaphoreType.DMA,
            pltpu.SemaphoreType.DMA,
        ],
    )
    def k(p_hbm, idx_hbm, out_hbm, idx_v, buf0, buf1, obuf, sem0, sem1):
        wid = lax.axis_index("s") * NC + lax.axis_index("c")
        pltpu.sync_copy(idx_hbm.at[wid], idx_v)

        pltpu.async_copy(p_hbm.at[idx_v.at[0]], buf0, sem0)
        pltpu.async_copy(p_hbm.at[idx_v.at[1]], buf1, sem1)

        def chunk(j, buf, sem):
            pltpu.make_async_copy(p_hbm.at[idx_v.at[j]], buf, sem).wait()

            def sample(s, _):
                def row(r, accs):
                    base = s * F + r
                    return tuple(
                        accs[v] + buf[base, pl.ds(L * v, L)] for v in range(NV)
                    )

                accs = lax.fori_loop(
                    0, F, row,
                    tuple(jnp.zeros((L,), jnp.float32) for _ in range(NV)),
                )
                orow = (j % OBJ) * CH + s
                for v in range(NV):
                    obuf[orow, pl.ds(L * v, L)] = accs[v]
                return 0

            lax.fori_loop(0, CH, sample, 0)

            @pl.when(j + 2 < NCH)
            def _():
                pltpu.async_copy(p_hbm.at[idx_v.at[j + 2]], buf, sem)

            @pl.when(j % OBJ == OBJ - 1)
            def _():
                blk = j // OBJ
                pltpu.sync_copy(obuf, out_hbm.at[pl.ds(wid * BPW + blk * OB, OB)])

        def pair(i2, _):
            chunk(i2 * 2, buf0, sem0)
            chunk(i2 * 2 + 1, buf1, sem1)
            return 0

        lax.fori_loop(0, NCH // 2, pair, 0)

    return k(p_flat, idx3)


BLK = 2048
GRID = B // BLK


def _stats1_body(acc_ref, xn_ref, w1n_ref, b1_ref, st_ref):
    t = acc_ref[...] + jnp.dot(
        xn_ref[...], w1n_ref[...], preferred_element_type=jnp.float32
    ) + b1_ref[...]

    @pl.when(pl.program_id(0) == 0)
    def _():
        st_ref[...] = jnp.zeros_like(st_ref)

    s = jnp.sum(t, axis=0, keepdims=True)
    s2 = jnp.sum(t * t, axis=0, keepdims=True)
    st_ref[...] += jnp.concatenate([s, s2], axis=0)


def _layer12_body(acc_ref, xn_ref, w1n_ref, b1_ref, g1_ref, be1_ref, st_ref,
                  w2_ref, b2_ref, h2_ref, st2_ref):
    t = acc_ref[...] + jnp.dot(
        xn_ref[...], w1n_ref[...], preferred_element_type=jnp.float32
    ) + b1_ref[...]
    m = st_ref[0:1, :] * (1.0 / B)
    var = st_ref[1:2, :] * (1.0 / B) - m * m
    h1 = jnp.maximum((t - m) / jnp.sqrt(var + EPS) * g1_ref[...] + be1_ref[...], 0.0)
    h2 = jnp.dot(h1, w2_ref[...], preferred_element_type=jnp.float32) + b2_ref[...]
    h2_ref[...] = h2

    @pl.when(pl.program_id(0) == 0)
    def _():
        st2_ref[...] = jnp.zeros_like(st2_ref)

    s = jnp.sum(h2, axis=0, keepdims=True)
    s2 = jnp.sum(h2 * h2, axis=0, keepdims=True)
    st2_ref[...] += jnp.concatenate([s, s2], axis=0)


def _head_body(h2_ref, st2_ref, g2_ref, be2_ref, w3_ref, b3_ref, xn_ref, o_ref):
    m = st2_ref[0:1, :] * (1.0 / B)
    var = st2_ref[1:2, :] * (1.0 / B) - m * m
    h = jnp.maximum(
        (h2_ref[...] - m) / jnp.sqrt(var + EPS) * g2_ref[...] + be2_ref[...], 0.0
    )
    o = jnp.dot(h, w3_ref[...], preferred_element_type=jnp.float32) + b3_ref[...]
    o_ref[...] = o + xn_ref[:, NUM - 1:NUM]


def _row_spec(width):
    return pl.BlockSpec((BLK, width), lambda i: (i, 0))


def _fixed_spec(r, c):
    return pl.BlockSpec((r, c), lambda i: (0, 0))


def _mlp_tail(acc, x_num, w1n, b1, g1, beta1, w2, b2, g2, beta2, w3, b3):
    st = pl.pallas_call(
        _stats1_body,
        grid=(GRID,),
        in_specs=[_row_spec(H1), _row_spec(NUM), _fixed_spec(NUM, H1),
                  _fixed_spec(1, H1)],
        out_specs=_fixed_spec(2, H1),
        out_shape=jax.ShapeDtypeStruct((2, H1), jnp.float32),
    )(acc, x_num, w1n, b1)

    h2, st2 = pl.pallas_call(
        _layer12_body,
        grid=(GRID,),
        in_specs=[_row_spec(H1), _row_spec(NUM), _fixed_spec(NUM, H1),
                  _fixed_spec(1, H1), _fixed_spec(1, H1), _fixed_spec(1, H1),
                  _fixed_spec(2, H1), _fixed_spec(H1, H2), _fixed_spec(1, H2)],
        out_specs=[_row_spec(H2), _fixed_spec(2, H2)],
        out_shape=[jax.ShapeDtypeStruct((B, H2), jnp.float32),
                   jax.ShapeDtypeStruct((2, H2), jnp.float32)],
    )(acc, x_num, w1n, b1, g1, beta1, st, w2, b2)

    out = pl.pallas_call(
        _head_body,
        grid=(GRID,),
        in_specs=[_row_spec(H2), _fixed_spec(2, H2), _fixed_spec(1, H2),
                  _fixed_spec(1, H2), _fixed_spec(H2, 1), _fixed_spec(1, 1),
                  _row_spec(NUM)],
        out_specs=_row_spec(1),
        out_shape=jax.ShapeDtypeStruct((B, 1), jnp.float32),
    )(h2, st2, g2, beta2, w3, b3, x_num)
    return out


def kernel(x_cat, x_num, tables, W1, b1, g1, beta1, W2, b2, g2, beta2, W3, b3):
    w1_emb = W1[: F * D].reshape(F, D, H1)
    w1n = W1[F * D:]

    p = _project_tables(tables, w1_emb).reshape(F * V, H1)

    offs = (jnp.arange(F, dtype=jnp.int32) * V)[None, :]
    idx3 = (x_cat.astype(jnp.int32) + offs).reshape(NW, NCH, CHF)
    acc = _sc_embed_bag(p, idx3)

    return _mlp_tail(
        acc, x_num, w1n,
        b1.reshape(1, H1), g1.reshape(1, H1), beta1.reshape(1, H1),
        W2, b2.reshape(1, H2), g2.reshape(1, H2), beta2.reshape(1, H2),
        W3, b3.reshape(1, 1),
    )
